# baseline (speedup 1.0000x reference)
import functools

import jax
import jax.numpy as jnp
from jax import lax
from jax.experimental import pallas as pl
from jax.experimental.pallas import tpu as pltpu
from jax.experimental.pallas import tpu_sc as plsc

BATCH = 16384
EMBED = 16
NC = 2
NS = 16
NW = NC * NS
RPW = BATCH // NW
CHUNK = 128
NCH = RPW // CHUNK
LANES = 16

_mesh = plsc.VectorSubcoreMesh(
    core_axis_name="c", subcore_axis_name="s", num_cores=NC, num_subcores=NS
)


@functools.partial(
    pl.kernel,
    out_type=(
        jax.ShapeDtypeStruct((NW, LANES), jnp.float32),
        jax.ShapeDtypeStruct((BATCH,), jnp.float32),
    ),
    mesh=_mesh,
    scratch_types=(
        pltpu.VMEM((NCH, CHUNK), jnp.int32),
        pltpu.VMEM((NCH, CHUNK), jnp.int32),
        pltpu.VMEM((RPW, EMBED), jnp.float32),
        pltpu.VMEM((RPW, EMBED), jnp.float32),
        pltpu.VMEM((RPW,), jnp.float32),
        pltpu.VMEM((RPW,), jnp.float32),
        pltpu.VMEM((RPW,), jnp.float32),
        pltpu.VMEM((LANES,), jnp.float32),
        pltpu.SemaphoreType.DMA,
    ),
    compiler_params=pltpu.CompilerParams(use_tc_tiling_on_sc=False),
)
def _gather_partial(
    uidx_hbm, midx_hbm, ue_hbm, me_hbm, ub_hbm, mb_hbm,
    partials_hbm, bsum_hbm,
    uidx_v, midx_v, urows_v, mrows_v, ub_v, mb_v, bs_v, acc_v, sem,
):
    wid = lax.axis_index("s") * NC + lax.axis_index("c")

    idx_cps = []
    for c in range(NCH):
        idx_cps.append(pltpu.async_copy(uidx_hbm.at[wid * NCH + c], uidx_v.at[c], sem))
        idx_cps.append(pltpu.async_copy(midx_hbm.at[wid * NCH + c], midx_v.at[c], sem))
    for cp in idx_cps:
        cp.wait()

    cps = []
    for c in range(NCH):
        sl = pl.ds(c * CHUNK, CHUNK)
        cps.append(pltpu.async_copy(ue_hbm.at[uidx_v.at[c]], urows_v.at[sl], sem))
        cps.append(pltpu.async_copy(me_hbm.at[midx_v.at[c]], mrows_v.at[sl], sem))
        cps.append(pltpu.async_copy(ub_hbm.at[uidx_v.at[c]], ub_v.at[sl], sem))
        cps.append(pltpu.async_copy(mb_hbm.at[midx_v.at[c]], mb_v.at[sl], sem))
    for cp in cps:
        cp.wait()

    def dot_body(i, acc):
        return acc + urows_v[i, :] * mrows_v[i, :]

    acc = lax.fori_loop(0, RPW, dot_body, jnp.zeros((LANES,), jnp.float32), unroll=8)
    acc_v[...] = acc
    pltpu.sync_copy(acc_v, partials_hbm.at[wid])

    def bias_body(j, carry):
        sl = pl.ds(j * LANES, LANES)
        bs_v[sl] = ub_v[sl] + mb_v[sl]
        return carry

    lax.fori_loop(0, RPW // LANES, bias_body, 0, unroll=4)
    pltpu.sync_copy(bs_v, bsum_hbm.at[pl.ds(wid * RPW, RPW)])


@functools.partial(
    pl.kernel,
    out_type=jax.ShapeDtypeStruct((BATCH,), jnp.float32),
    mesh=_mesh,
    scratch_types=(
        pltpu.VMEM((NW, LANES), jnp.float32),
        pltpu.VMEM((RPW,), jnp.float32),
        pltpu.VMEM((RPW,), jnp.float32),
    ),
    compiler_params=pltpu.CompilerParams(
        use_tc_tiling_on_sc=False, needs_layout_passes=False
    ),
)
def _finalize(partials_hbm, bsum_hbm, out_hbm, part_v, b_v, o_v):
    wid = lax.axis_index("s") * NC + lax.axis_index("c")
    pltpu.sync_copy(partials_hbm, part_v)
    pltpu.sync_copy(bsum_hbm.at[pl.ds(wid * RPW, RPW)], b_v)

    acc = part_v[0, :]
    for i in range(1, NW):
        acc = acc + part_v[i, :]
    s = jnp.sum(acc)

    def sig_body(j, carry):
        sl = pl.ds(j * LANES, LANES)
        x = s + b_v[sl]
        o_v[sl] = 1.0 / (1.0 + jnp.exp(-x))
        return carry

    lax.fori_loop(0, RPW // LANES, sig_body, 0, unroll=4)
    pltpu.sync_copy(o_v, out_hbm.at[pl.ds(wid * RPW, RPW)])


def kernel(inputs, user_embedding, movie_embedding, user_bias, movie_bias):
    uidx = inputs[:, 0].reshape(NW * NCH, CHUNK)
    midx = inputs[:, 1].reshape(NW * NCH, CHUNK)
    ub = user_bias.reshape(-1)
    mb = movie_bias.reshape(-1)
    partials, bsum = _gather_partial(
        uidx, midx, user_embedding, movie_embedding, ub, mb
    )
    out = _finalize(partials, bsum)
    return out.reshape(BATCH, 1)


# speedup vs baseline: 6.0994x; 6.0994x over previous
import functools

import jax
import jax.numpy as jnp
from jax import lax
from jax.experimental import pallas as pl
from jax.experimental.pallas import tpu as pltpu
from jax.experimental.pallas import tpu_sc as plsc

BATCH = 16384
EMBED = 16
NC = 2
NS = 16
NW = NC * NS
RPW = BATCH // NW
CHUNK = 128
NCH = RPW // CHUNK
LANES = 16
GRP = 8
NSUP = RPW // (2 * GRP)

_mesh = plsc.VectorSubcoreMesh(
    core_axis_name="c", subcore_axis_name="s", num_cores=NC, num_subcores=NS
)

_slab = pltpu.VMEM((LANES, 128), jnp.float32)


@functools.partial(
    pl.kernel,
    out_type=jax.ShapeDtypeStruct((NW * 128,), jnp.float32),
    mesh=_mesh,
    scratch_types=(
        pltpu.VMEM((RPW + LANES,), jnp.int32),
        pltpu.VMEM((RPW + LANES,), jnp.int32),
        ((_slab,) * GRP, (_slab,) * GRP),
        ((_slab,) * GRP, (_slab,) * GRP),
        pltpu.VMEM((128,), jnp.float32),
        ((pltpu.SemaphoreType.DMA,) * GRP,) * 2,
    ),
    compiler_params=pltpu.CompilerParams(needs_layout_passes=False),
)
def _dot_partial(
    ueT_hbm, meT_hbm, uidx_hbm, midx_hbm,
    partials_hbm,
    uidx_v, midx_v, uslabs, mslabs, stage_v, sems,
):
    wid = lax.axis_index("s") * NC + lax.axis_index("c")
    base = pl.multiple_of(wid * RPW, 128)
    pltpu.sync_copy(uidx_hbm.at[pl.ds(base, RPW)], uidx_v.at[pl.ds(0, RPW)])
    pltpu.sync_copy(midx_hbm.at[pl.ds(base, RPW)], midx_v.at[pl.ds(0, RPW)])

    rows = lax.iota(jnp.int32, LANES)

    def idx_vecs(jbase):
        sel = rows + jnp.full((LANES,), jbase, jnp.int32)
        uvec = plsc.load_gather(uidx_v, [sel])
        mvec = plsc.load_gather(midx_v, [sel])
        return uvec, mvec

    def fire(bank, jbase):
        uvec, mvec = idx_vecs(jbase)
        for b in range(GRP):
            offu = pl.multiple_of(((uvec[b] >> 7) * 128).astype(jnp.int32), 128)
            offm = pl.multiple_of(((mvec[b] >> 7) * 128).astype(jnp.int32), 128)
            pltpu.async_copy(
                ueT_hbm.at[:, pl.ds(offu, 128)], uslabs[bank][b], sems[bank][b]
            )
            pltpu.async_copy(
                meT_hbm.at[:, pl.ds(offm, 128)], mslabs[bank][b], sems[bank][b]
            )

    def consume(bank, jbase, acc):
        uvec, mvec = idx_vecs(jbase)
        ulane = uvec & 127
        mlane = mvec & 127
        for b in range(GRP):
            pltpu.make_async_copy(
                ueT_hbm.at[:, pl.ds(0, 128)], uslabs[bank][b], sems[bank][b]
            ).wait()
            pltpu.make_async_copy(
                meT_hbm.at[:, pl.ds(0, 128)], mslabs[bank][b], sems[bank][b]
            ).wait()
            lu = jnp.full((LANES,), ulane[b], jnp.int32)
            lm = jnp.full((LANES,), mlane[b], jnp.int32)
            ucol = plsc.load_gather(uslabs[bank][b], [rows, lu])
            mcol = plsc.load_gather(mslabs[bank][b], [rows, lm])
            acc = acc + ucol * mcol
        return acc

    fire(0, 0)
    fire(1, GRP)

    def super_body(h, acc):
        jb = h * 2 * GRP
        acc = consume(0, jb, acc)

        @pl.when(h < NSUP - 1)
        def _():
            fire(0, jb + 2 * GRP)

        acc = consume(1, jb + GRP, acc)

        @pl.when(h < NSUP - 1)
        def _():
            fire(1, jb + 3 * GRP)

        return acc

    acc = lax.fori_loop(0, NSUP, super_body, jnp.zeros((LANES,), jnp.float32))
    plsc.store_scatter(stage_v, [rows], acc)
    pltpu.sync_copy(
        stage_v, partials_hbm.at[pl.ds(pl.multiple_of(wid * 128, 128), 128)]
    )


@functools.partial(
    pl.kernel,
    out_type=jax.ShapeDtypeStruct((BATCH,), jnp.float32),
    mesh=_mesh,
    scratch_types=(
        pltpu.VMEM((NCH, CHUNK), jnp.int32),
        pltpu.VMEM((NCH, CHUNK), jnp.int32),
        pltpu.VMEM((RPW,), jnp.float32),
        pltpu.VMEM((RPW,), jnp.float32),
        pltpu.VMEM((NW * 128,), jnp.float32),
        pltpu.VMEM((RPW,), jnp.float32),
        pltpu.SemaphoreType.DMA,
    ),
    compiler_params=pltpu.CompilerParams(
        use_tc_tiling_on_sc=False, needs_layout_passes=False
    ),
)
def _finalize(
    uidx_hbm, midx_hbm, ub_hbm, mb_hbm, partials_hbm,
    out_hbm,
    uidx_v, midx_v, ub_v, mb_v, part_v, o_v, sem,
):
    wid = lax.axis_index("s") * NC + lax.axis_index("c")

    idx_cps = []
    for c in range(NCH):
        idx_cps.append(pltpu.async_copy(uidx_hbm.at[wid * NCH + c], uidx_v.at[c], sem))
        idx_cps.append(pltpu.async_copy(midx_hbm.at[wid * NCH + c], midx_v.at[c], sem))
    idx_cps.append(pltpu.async_copy(partials_hbm, part_v, sem))
    for cp in idx_cps:
        cp.wait()

    cps = []
    for c in range(NCH):
        sl = pl.ds(c * CHUNK, CHUNK)
        cps.append(pltpu.async_copy(ub_hbm.at[uidx_v.at[c]], ub_v.at[sl], sem))
        cps.append(pltpu.async_copy(mb_hbm.at[midx_v.at[c]], mb_v.at[sl], sem))
    for cp in cps:
        cp.wait()

    acc = part_v[pl.ds(0, LANES)]
    for w in range(1, NW):
        acc = acc + part_v[pl.ds(w * 128, LANES)]
    s = jnp.sum(acc)

    def sig_body(k, carry):
        sl = pl.ds(k * LANES, LANES)
        x = s + ub_v[sl] + mb_v[sl]
        o_v[sl] = 1.0 / (1.0 + jnp.exp(-x))
        return carry

    lax.fori_loop(0, RPW // LANES, sig_body, 0, unroll=4)
    pltpu.sync_copy(o_v, out_hbm.at[pl.ds(wid * RPW, RPW)])


def kernel(inputs, user_embedding, movie_embedding, user_bias, movie_bias):
    uidx = inputs[:, 0]
    midx = inputs[:, 1]
    uidx2 = uidx.reshape(NW * NCH, CHUNK)
    midx2 = midx.reshape(NW * NCH, CHUNK)
    ub = user_bias.reshape(-1)
    mb = movie_bias.reshape(-1)
    partials = _dot_partial(user_embedding.T, movie_embedding.T, uidx, midx)
    out = _finalize(uidx2, midx2, ub, mb, partials)
    return out.reshape(BATCH, 1)
